# baseline (speedup 1.0000x reference)
import functools
import math

import jax
import jax.numpy as jnp
from jax import lax
from jax.experimental import pallas as pl
from jax.experimental.pallas import tpu as pltpu
from jax.experimental.pallas import tpu_sc as plsc

N = 10000
E = 320000
G = 16
NPG = 625
NPP = 640
H = 128
EPS = 1e-5
K_LIST = [313, 157, 79, 40]

GSZ = NPP * NPP
CHUNK = 4 * GSZ
TRASH = 8192
REGION = CHUNK + TRASH
OSLICE = CHUNK // 16
ZBUF = 3200
TSLICE = TRASH // 16

EPT = E // 16
CE = 4096
NCH = 5
CE_TAIL = EPT - (NCH - 1) * CE
CROWS = CE // 128
MAGIC = 26844


def _sc_build_adjacency(edge_flat, half):
  mesh = plsc.VectorSubcoreMesh(core_axis_name="c", subcore_axis_name="s")

  @functools.partial(
      pl.kernel,
      out_type=jax.ShapeDtypeStruct((G * GSZ // 2,), jnp.float32),
      mesh=mesh,
      scratch_types=[
          pltpu.VMEM((2, 2 * CE), jnp.int32),
          pltpu.VMEM((2, CROWS, 128), jnp.int32),
          pltpu.VMEM((128,), jnp.float32),
          pltpu.VMEM((ZBUF,), jnp.float32),
          pltpu.VMEM_SHARED((REGION,), jnp.float32),
          pltpu.SemaphoreType.DMA,
          pltpu.SemaphoreType.DMA,
      ],
  )
  def build(edge_hbm, out_hbm, edge_c, idx_c,
            ones_v, zeros_v, acc_sh, sem, esem):
    c = lax.axis_index("c")
    s = lax.axis_index("s")
    lane = lax.iota(jnp.int32, 16)

    for i in range(8):
      ones_v[pl.ds(i * 16, 16)] = jnp.ones((16,), jnp.float32)

    def zfill(i, _):
      zeros_v[pl.ds(i * 16, 16)] = jnp.zeros((16,), jnp.float32)
      return ()
    lax.fori_loop(0, ZBUF // 16, zfill, ())

    def zero_own_slices():
      for z in range(32):
        pltpu.sync_copy(zeros_v, acc_sh.at[pl.ds(s * OSLICE + z * ZBUF, ZBUF)])
      pltpu.sync_copy(zeros_v.at[pl.ds(0, TSLICE)],
                      acc_sh.at[pl.ds(CHUNK + s * TSLICE, TSLICE)])

    start = s * EPT
    astart = (start >> 7) << 7
    head = start - astart
    FB = [0, CE, 2 * CE, 3 * CE, 16000]

    def fetch(ci, eb):
      off = pl.multiple_of(astart + FB[ci], 128)
      return [
          pltpu.async_copy(edge_hbm.at[:, pl.ds(off, CE)],
                           edge_c.at[:, pl.ds(eb * CE, CE)], esem),
      ]

    zero_own_slices()
    plsc.subcore_barrier()

    for p in range(1):
      chunk = half * 2 + c
      base_flat = chunk * CHUNK

      efetch = {0: fetch(0, 0)}
      sdescs = {}
      for ci in range(NCH):
        eb = ci % 2
        fb = FB[ci]
        for d in efetch.pop(ci % 2):
          d.wait()
        if ci + 1 < NCH:
          efetch[(ci + 1) % 2] = fetch(ci + 1, (ci + 1) % 2)

        for d in sdescs.pop(eb, []):
          d.wait()

        def idx_body(jj, _):
          for u in range(4):
            j = jj * 4 + u
            e = j * 16
            sv = edge_c[0, pl.ds(eb * CE + e, 16)]
            dv = edge_c[1, pl.ds(eb * CE + e, 16)]
            g = (dv * MAGIC) >> 24
            ld = dv - g * NPG
            ls = sv - ((sv * MAGIC) >> 24) * NPG
            tiled = ((ls >> 7) * 81920 + ((ld >> 3) << 10)
                     + ((ld & 7) << 7) + (ls & 127))
            flat = g * GSZ + tiled
            loc = flat - base_flat
            rel = fb + e + lane
            inb = (loc >= 0) & (loc < CHUNK)
            if ci == 0:
              inb &= rel >= head
            if ci == NCH - 1:
              inb &= (rel >= 4 * CE) & (rel < head + EPT)
            tr = CHUNK + ((rel + s * 1280) & (TRASH - 1))
            idx_c[eb, j // 8, pl.ds((j % 8) * 16, 16)] = jnp.where(inb, loc, tr)
          return ()
        lax.fori_loop(0, CE // 64, idx_body, ())

        sdescs[eb] = [
            pltpu.async_copy(ones_v, acc_sh.at[idx_c.at[eb, j]], sem, add=True)
            for j in range(CROWS)
        ]
      for descs in sdescs.values():
        for d in descs:
          d.wait()
      plsc.subcore_barrier()

      pltpu.sync_copy(
          acc_sh.at[pl.ds(s * OSLICE, OSLICE)],
          out_hbm.at[pl.ds(c * CHUNK + s * OSLICE, OSLICE)],
      )

  return build(edge_flat)


def _tc_forward_body(A_ref, x_ref, *refs):
  o_ref = refs[-1]
  wrefs = refs[:-1]
  f32 = jnp.float32
  A5 = A_ref[0]
  Ac = [A5[tc].reshape(NPP, H) for tc in range(5)]
  h = x_ref[0]

  def matA(u):
    acc = jnp.dot(Ac[0], u[0:H], preferred_element_type=f32)
    for tc in range(1, 5):
      acc = acc + jnp.dot(Ac[tc], u[tc * H:(tc + 1) * H],
                          preferred_element_type=f32)
    return acc

  rowi = lax.broadcasted_iota(jnp.int32, (NPP, 1), 0)
  colj = lax.broadcasted_iota(jnp.int32, (1, NPP), 1)
  tri = colj < rowi
  ident = (lax.broadcasted_iota(jnp.int32, (NPP, NPP), 0) ==
           lax.broadcasted_iota(jnp.int32, (NPP, NPP), 1)).astype(f32)
  m = (rowi < NPG).astype(f32)
  bn_scale = 1.0 / math.sqrt(1.0 + EPS)

  for st in range(4):
    k = K_LIST[st]
    (Wc_r, Wf_r, bc_r, bf_r, gg_r, be_r, ws_r, bs_r) = wrefs[st * 8:st * 8 + 8]
    Wc = Wc_r[...]
    Wf = Wf_r[...]
    bc = bc_r[...]
    bf = bf_r[...]
    gg = gg_r[...]
    be = be_r[...]
    ws = ws_r[...]
    bs = bs_r[0, 0]

    degv = matA(m)
    deg = m * (degv + 1.0)
    dinv = m * lax.rsqrt(deg + (1.0 - m))

    hw = jnp.dot(h, Wc, preferred_element_type=f32)
    v = matA(dinv * hw)
    conv = dinv * v + (dinv * dinv) * hw + bc

    h2 = jnp.maximum(jnp.dot(conv, Wf, preferred_element_type=f32) + bf, 0.0)
    h2 = h2 * (bn_scale * gg) + be

    hs = jnp.sum(h2 * ws, axis=1, keepdims=True)
    vs = matA(dinv * hs)
    sc = jnp.tanh(dinv * vs + (dinv * dinv) * hs + bs)

    sm = jnp.where(m > 0, sc, -2.0)
    smT = lax.dot_general(sm, ident, (((0,), (0,)), ((), ())))
    cmp = (smT > sm) | ((smT == sm) & tri)
    rank = jnp.sum(cmp.astype(f32), axis=1, keepdims=True)
    m = (rank < k).astype(f32)
    h = h2 * sc * m

  sums = jnp.sum(h, axis=0, keepdims=True)
  cnt = jnp.sum(m)
  o_ref[0] = sums / cnt


def _tc_forward(A3, xp, weights):
  ng = A3.shape[0]
  wspecs = []
  for _ in range(4):
    wspecs += [
        pl.BlockSpec((H, H), lambda g: (0, 0)),
        pl.BlockSpec((H, H), lambda g: (0, 0)),
    ] + [pl.BlockSpec((1, H), lambda g: (0, 0))] * 6
  return pl.pallas_call(
      _tc_forward_body,
      grid=(ng,),
      in_specs=[
          pl.BlockSpec((1, 5, NPP // 8, 8, H), lambda g: (g, 0, 0, 0, 0)),
          pl.BlockSpec((1, NPP, H), lambda g: (g, 0, 0)),
      ] + wspecs,
      out_specs=pl.BlockSpec((1, 1, H), lambda g: (g, 0, 0)),
      out_shape=jax.ShapeDtypeStruct((ng, 1, H), jnp.float32),
  )(A3, xp, *weights)


def kernel(x, edge_index, batch,
           Wc1, bc1, Wf1, bf1, g1, be1, Ws1, bs1,
           Wc2, bc2, Wf2, bf2, g2, be2, Ws2, bs2,
           Wc3, bc3, Wf3, bf3, g3, be3, Ws3, bs3,
           Wc4, bc4, Wf4, bf4, g4, be4, Ws4, bs4):
  del batch
  xp = jnp.pad(x.reshape(G, NPG, H), ((0, 0), (0, NPP - NPG), (0, 0)))

  weights = []
  for (Wc, bc, Wf, bf, g_, be, Ws, bs) in (
      (Wc1, bc1, Wf1, bf1, g1, be1, Ws1, bs1),
      (Wc2, bc2, Wf2, bf2, g2, be2, Ws2, bs2),
      (Wc3, bc3, Wf3, bf3, g3, be3, Ws3, bs3),
      (Wc4, bc4, Wf4, bf4, g4, be4, Ws4, bs4)):
    weights += [Wc, Wf, bc.reshape(1, H), bf.reshape(1, H),
                g_.reshape(1, H), be.reshape(1, H), Ws.reshape(1, H),
                jnp.broadcast_to(bs, (1, H))]

  outs = []
  for half in range(2):
    A_flat = _sc_build_adjacency(edge_index, half)
    A3 = A_flat.reshape(G // 2, 5, NPP // 8, 8, H)
    outs.append(_tc_forward(A3, xp[half * 8:half * 8 + 8], weights))
  return jnp.concatenate(outs, axis=0).reshape(G, H)
